# baseline (speedup 1.0000x reference)
import functools

import jax
import jax.numpy as jnp
from jax.experimental import pallas as pl
from jax.experimental.pallas import tpu as pltpu
from jax.experimental.pallas import tpu_sc as plsc

_NC = 2
_NS = 16
_LANES = 16

_SC_BATCHES = 32
_ROWS = 16
_RG = 4
_BG = 8


def _sc_body(p_hbm, t_hbm, o_hbm, *, batch0):
    P, D = t_hbm.shape
    n_tiles = _NC * _NS
    n_row_blocks = P // _ROWS
    rpg = n_row_blocks // _RG
    bpg = _SC_BATCHES // _BG

    def block_body(p_v, t_v, o_v):
        @pl.loop(0, _ROWS)
        def _row(r):
            @pl.loop(0, D, step=_LANES, unroll=8)
            def _col(c):
                o_v[0, r, pl.ds(c, _LANES)] = (
                    p_v[0, r, pl.ds(c, _LANES)] + t_v[r, pl.ds(c, _LANES)]
                )

    pltpu.emit_pipeline(
        block_body,
        grid=(n_tiles, rpg, bpg),
        in_specs=[
            pl.BlockSpec(
                (1, _ROWS, D),
                lambda g, i, j: (batch0 + (g % _BG) * bpg + j, (g // _BG) * rpg + i, 0),
            ),
            pl.BlockSpec((_ROWS, D), lambda g, i, j: ((g // _BG) * rpg + i, 0)),
        ],
        out_specs=[
            pl.BlockSpec(
                (1, _ROWS, D),
                lambda g, i, j: ((g % _BG) * bpg + j, (g // _BG) * rpg + i, 0),
            )
        ],
        core_axis_name=("c", "s"),
        dimension_semantics=(pltpu.PARALLEL, pltpu.ARBITRARY, pltpu.ARBITRARY),
    )(p_hbm, t_hbm, o_hbm)


def _tc_add(p_ref, t_ref, o_ref):
    o_ref[...] = p_ref[...] + t_ref[...]


def kernel(patches, pos_table):
    B, P, D = patches.shape
    b_tc = B - _SC_BATCHES
    bb = 8
    out_full = pl.pallas_call(
        _tc_add,
        grid=(b_tc // bb,),
        in_specs=[
            pl.BlockSpec((bb, P, D), lambda b: (b, 0, 0)),
            pl.BlockSpec((P, D), lambda b: (0, 0)),
        ],
        out_specs=pl.BlockSpec((bb, P, D), lambda b: (b, 0, 0)),
        out_shape=jax.ShapeDtypeStruct((B, P, D), patches.dtype),
        compiler_params=pltpu.CompilerParams(vmem_limit_bytes=64 * 1024 * 1024),
    )(patches, pos_table)

    mesh = plsc.VectorSubcoreMesh(core_axis_name="c", subcore_axis_name="s")
    sc_add = pl.kernel(
        functools.partial(_sc_body, batch0=b_tc),
        out_type=jax.ShapeDtypeStruct((_SC_BATCHES, P, D), patches.dtype),
        mesh=mesh,
    )
    out_sc = sc_add(patches, pos_table)
    return jax.lax.dynamic_update_slice(out_full, out_sc, (b_tc, 0, 0))


# speedup vs baseline: 1.5413x; 1.5413x over previous
import jax
from jax.experimental import pallas as pl
from jax.experimental.pallas import tpu as pltpu


def _add_kernel(p_ref, t_ref, o_ref):
    o_ref[...] = p_ref[...] + t_ref[...]


def kernel(patches, pos_table):
    B, P, D = patches.shape
    bb = 8
    return pl.pallas_call(
        _add_kernel,
        grid=(B // bb,),
        in_specs=[
            pl.BlockSpec((bb, P, D), lambda b: (b, 0, 0)),
            pl.BlockSpec((P, D), lambda b: (0, 0)),
        ],
        out_specs=pl.BlockSpec((bb, P, D), lambda b: (b, 0, 0)),
        out_shape=jax.ShapeDtypeStruct((B, P, D), patches.dtype),
        compiler_params=pltpu.CompilerParams(vmem_limit_bytes=64 * 1024 * 1024),
    )(patches, pos_table)
